# baseline (speedup 1.0000x reference)
import functools

import jax
import jax.numpy as jnp
from jax.experimental import pallas as pl
from jax.experimental.pallas import tpu as pltpu

B, N, IN_F, OUT_F = 16, 1024, 64, 64
SENS = 0.05


def _gnn_batch_kernel(x_ref, a_ref, wp_ref, bp_ref, v_ref, w1_ref, b1_ref,
                      out_ref):
    x = x_ref[0]
    a = a_ref[0]

    p = jax.lax.dot_general(x, wp_ref[...], (((1,), (0,)), ((), ())),
                            preferred_element_type=jnp.float32,
                            precision=jax.lax.Precision.DEFAULT) + bp_ref[...]
    t2 = p[:, 0:OUT_F]
    t3 = p[:, OUT_F:2 * OUT_F]
    t4 = p[:, 2 * OUT_F:3 * OUT_F]

    u_row = jax.lax.dot_general(v_ref[:, :IN_F], x, (((1,), (1,)), ((), ())),
                                preferred_element_type=jnp.float32,
                                precision=jax.lax.Precision.HIGHEST) \
        + v_ref[:, IN_F:IN_F + 1]

    d = jax.lax.dot_general(t4, t3, (((1,), (1,)), ((), ())),
                            preferred_element_type=jnp.float32,
                            precision=jax.lax.Precision.DEFAULT)

    mask = (a < SENS) & (a > 0)
    neg_inf = jnp.float32(-jnp.inf)
    sm = jnp.where(mask, d + a * u_row, neg_inf)
    smax = jnp.max(sm, axis=0, keepdims=True)
    m = jnp.maximum(smax, jnp.float32(0.0))
    e = jnp.exp2(sm - m)
    denom = jnp.sum(e, axis=0, keepdims=True)
    invd = jnp.float32(1.0) / (denom + jnp.float32(1e-16))

    g = jax.lax.dot_general(e, t2, (((0,), (0,)), ((), ())),
                            preferred_element_type=jnp.float32,
                            precision=jax.lax.Precision.DEFAULT)
    contrib = jax.lax.dot_general(invd, g, (((1,), (0,)), ((), ())),
                                  preferred_element_type=jnp.float32,
                                  precision=jax.lax.Precision.HIGHEST)

    ones_row = jnp.full((1, N), jnp.float32(1.0) / jnp.float32(N))
    mean_x = jax.lax.dot_general(ones_row, x, (((1,), (0,)), ((), ())),
                                 preferred_element_type=jnp.float32,
                                 precision=jax.lax.Precision.HIGHEST)
    lin = jax.lax.dot_general(mean_x, w1_ref[...], (((1,), (1,)), ((), ())),
                              preferred_element_type=jnp.float32,
                              precision=jax.lax.Precision.HIGHEST) + b1_ref[...]
    out_ref[0, 0] = lin[0] + contrib[0] * (jnp.float32(1.0) / jnp.float32(N))


def kernel(node_obs, adj, W1, b1, W2, b2, W3, b3, W4, b4, W5, b5):
    log2e = jnp.float32(1.4426950408889634)
    ls = log2e / jnp.sqrt(jnp.float32(OUT_F))
    w5c = W5[:, 0]
    v = (W3.T @ w5c) * ls
    c = jnp.dot(b3, w5c) * ls
    wp = jnp.concatenate([W2.T, W3.T, W4.T * ls], axis=1)
    bp = jnp.concatenate([b2, b3, b4 * ls])[None, :]
    vc = jnp.concatenate([v, jnp.full((1,), c, jnp.float32),
                          jnp.zeros((IN_F - 1,), jnp.float32)])[None, :]

    grid_spec = pl.GridSpec(
        grid=(B,),
        in_specs=[
            pl.BlockSpec((1, N, IN_F), lambda b: (b, 0, 0)),
            pl.BlockSpec((1, N, N), lambda b: (b, 0, 0)),
            pl.BlockSpec((IN_F, 3 * OUT_F), lambda b: (0, 0)),
            pl.BlockSpec((1, 3 * OUT_F), lambda b: (0, 0)),
            pl.BlockSpec((1, 2 * IN_F), lambda b: (0, 0)),
            pl.BlockSpec((OUT_F, IN_F), lambda b: (0, 0)),
            pl.BlockSpec((1, OUT_F), lambda b: (0, 0)),
        ],
        out_specs=pl.BlockSpec((1, 1, OUT_F), lambda b: (b, 0, 0)),
    )

    out = pl.pallas_call(
        _gnn_batch_kernel,
        grid_spec=grid_spec,
        out_shape=jax.ShapeDtypeStruct((B, 1, OUT_F), jnp.float32),
    )(node_obs, adj, wp, bp, vc, W1, b1[None, :])
    return out.reshape(B, OUT_F)


# speedup vs baseline: 1.1170x; 1.1170x over previous
import functools

import jax
import jax.numpy as jnp
from jax.experimental import pallas as pl
from jax.experimental.pallas import tpu as pltpu

B, N, IN_F, OUT_F = 16, 1024, 64, 64
SENS = 0.05


def _gnn_batch_kernel(x_ref, a_ref, wp_ref, bp_ref, v_ref, w1_ref, b1_ref,
                      out_ref):
    x = x_ref[0]
    a = a_ref[0]

    p = jax.lax.dot_general(x, wp_ref[...], (((1,), (0,)), ((), ())),
                            preferred_element_type=jnp.float32,
                            precision=jax.lax.Precision.DEFAULT) + bp_ref[...]
    t2 = p[:, 0:OUT_F]
    t3 = p[:, OUT_F:2 * OUT_F]
    t4 = p[:, 2 * OUT_F:3 * OUT_F]

    u_row = jax.lax.dot_general(v_ref[:, :IN_F], x, (((1,), (1,)), ((), ())),
                                preferred_element_type=jnp.float32,
                                precision=jax.lax.Precision.HIGHEST) \
        + v_ref[:, IN_F:IN_F + 1]

    d = jax.lax.dot_general(t4, t3, (((1,), (1,)), ((), ())),
                            preferred_element_type=jnp.float32,
                            precision=jax.lax.Precision.DEFAULT)

    mask = (a < SENS) & (a > 0)
    neg_inf = jnp.float32(-jnp.inf)
    sm = jnp.where(mask, d + a * u_row, neg_inf)
    smax = jnp.max(sm, axis=0, keepdims=True)
    m = jnp.maximum(smax, jnp.float32(0.0))
    e = jnp.exp2(sm - m)
    denom = jnp.sum(e, axis=0, keepdims=True)
    invd = jnp.float32(1.0) / (denom + jnp.float32(1e-16))

    g = jax.lax.dot_general(e, t2, (((0,), (0,)), ((), ())),
                            preferred_element_type=jnp.float32,
                            precision=jax.lax.Precision.DEFAULT)
    contrib = jax.lax.dot_general(invd, g, (((1,), (0,)), ((), ())),
                                  preferred_element_type=jnp.float32,
                                  precision=jax.lax.Precision.HIGHEST)

    mean_x = jnp.mean(x, axis=0, keepdims=True)
    lin = jax.lax.dot_general(mean_x, w1_ref[...], (((1,), (1,)), ((), ())),
                              preferred_element_type=jnp.float32,
                              precision=jax.lax.Precision.HIGHEST) + b1_ref[...]
    out_ref[0, 0] = lin[0] + contrib[0] * (jnp.float32(1.0) / jnp.float32(N))


def kernel(node_obs, adj, W1, b1, W2, b2, W3, b3, W4, b4, W5, b5):
    log2e = jnp.float32(1.4426950408889634)
    ls = log2e / jnp.sqrt(jnp.float32(OUT_F))
    w5c = W5[:, 0]
    v = (W3.T @ w5c) * ls
    c = jnp.dot(b3, w5c) * ls
    wp = jnp.concatenate([W2.T, W3.T, W4.T * ls], axis=1)
    bp = jnp.concatenate([b2, b3, b4 * ls])[None, :]
    vc = jnp.concatenate([v, jnp.full((1,), c, jnp.float32),
                          jnp.zeros((IN_F - 1,), jnp.float32)])[None, :]

    grid_spec = pl.GridSpec(
        grid=(B,),
        in_specs=[
            pl.BlockSpec((1, N, IN_F), lambda b: (b, 0, 0)),
            pl.BlockSpec((1, N, N), lambda b: (b, 0, 0)),
            pl.BlockSpec((IN_F, 3 * OUT_F), lambda b: (0, 0)),
            pl.BlockSpec((1, 3 * OUT_F), lambda b: (0, 0)),
            pl.BlockSpec((1, 2 * IN_F), lambda b: (0, 0)),
            pl.BlockSpec((OUT_F, IN_F), lambda b: (0, 0)),
            pl.BlockSpec((1, OUT_F), lambda b: (0, 0)),
        ],
        out_specs=pl.BlockSpec((1, 1, OUT_F), lambda b: (b, 0, 0)),
    )

    out = pl.pallas_call(
        _gnn_batch_kernel,
        grid_spec=grid_spec,
        out_shape=jax.ShapeDtypeStruct((B, 1, OUT_F), jnp.float32),
    )(node_obs, adj, wp, bp, vc, W1, b1[None, :])
    return out.reshape(B, OUT_F)


# speedup vs baseline: 1.2439x; 1.1137x over previous
import functools

import jax
import jax.numpy as jnp
from jax.experimental import pallas as pl
from jax.experimental.pallas import tpu as pltpu

B, N, IN_F, OUT_F = 16, 1024, 64, 64
SENS = 0.05


def _gnn_batch_kernel(x_ref, a_ref, wp_ref, bp_ref, v_ref, w1_ref, b1_ref,
                      out_ref):
    x = x_ref[0]
    a = a_ref[0]

    p = jax.lax.dot_general(x, wp_ref[...], (((1,), (0,)), ((), ())),
                            preferred_element_type=jnp.float32,
                            precision=jax.lax.Precision.DEFAULT) + bp_ref[...]
    t2a = p[:, 0:2 * OUT_F]
    t3 = p[:, 2 * OUT_F:3 * OUT_F]
    t4 = p[:, 3 * OUT_F:4 * OUT_F]

    u_row = jax.lax.dot_general(v_ref[:, :IN_F], x, (((1,), (1,)), ((), ())),
                                preferred_element_type=jnp.float32,
                                precision=jax.lax.Precision.DEFAULT) \
        + v_ref[:, IN_F:IN_F + 1]

    d = jax.lax.dot_general(t4, t3, (((1,), (1,)), ((), ())),
                            preferred_element_type=jnp.float32,
                            precision=jax.lax.Precision.DEFAULT)

    mask = (a < SENS) & (a > 0)
    neg_inf = jnp.float32(-jnp.inf)
    sm = jnp.where(mask, d + a * u_row, neg_inf)
    smax = jnp.max(sm, axis=0, keepdims=True)
    m = jnp.maximum(smax, jnp.float32(0.0))
    e = jnp.exp2(sm - m)

    ga = jax.lax.dot_general(e, t2a, (((0,), (0,)), ((), ())),
                             preferred_element_type=jnp.float32,
                             precision=jax.lax.Precision.DEFAULT)
    denom_col = ga[:, OUT_F:OUT_F + 1]
    invd_col = jnp.float32(1.0) / (denom_col + jnp.float32(1e-16))
    contrib = jax.lax.dot_general(invd_col, ga[:, 0:OUT_F],
                                  (((0,), (0,)), ((), ())),
                                  preferred_element_type=jnp.float32,
                                  precision=jax.lax.Precision.DEFAULT)

    mean_x = jnp.mean(x, axis=0, keepdims=True)
    lin = jax.lax.dot_general(mean_x, w1_ref[...], (((1,), (1,)), ((), ())),
                              preferred_element_type=jnp.float32,
                              precision=jax.lax.Precision.HIGHEST) + b1_ref[...]
    out_ref[0, 0] = lin[0] + contrib[0] * (jnp.float32(1.0) / jnp.float32(N))


def kernel(node_obs, adj, W1, b1, W2, b2, W3, b3, W4, b4, W5, b5):
    log2e = jnp.float32(1.4426950408889634)
    ls = log2e / jnp.sqrt(jnp.float32(OUT_F))
    w5c = W5[:, 0]
    v = (W3.T @ w5c) * ls
    c = jnp.dot(b3, w5c) * ls
    zcols = jnp.zeros((IN_F, OUT_F), jnp.float32)
    wp = jnp.concatenate([W2.T, zcols, W3.T, W4.T * ls], axis=1)
    ones_bias = jnp.zeros((OUT_F,), jnp.float32).at[0].set(1.0)
    bp = jnp.concatenate([b2, ones_bias, b3, b4 * ls])[None, :]
    vc = jnp.concatenate([v, jnp.full((1,), c, jnp.float32),
                          jnp.zeros((IN_F - 1,), jnp.float32)])[None, :]

    grid_spec = pl.GridSpec(
        grid=(B,),
        in_specs=[
            pl.BlockSpec((1, N, IN_F), lambda b: (b, 0, 0)),
            pl.BlockSpec((1, N, N), lambda b: (b, 0, 0)),
            pl.BlockSpec((IN_F, 4 * OUT_F), lambda b: (0, 0)),
            pl.BlockSpec((1, 4 * OUT_F), lambda b: (0, 0)),
            pl.BlockSpec((1, 2 * IN_F), lambda b: (0, 0)),
            pl.BlockSpec((OUT_F, IN_F), lambda b: (0, 0)),
            pl.BlockSpec((1, OUT_F), lambda b: (0, 0)),
        ],
        out_specs=pl.BlockSpec((1, 1, OUT_F), lambda b: (b, 0, 0)),
    )

    out = pl.pallas_call(
        _gnn_batch_kernel,
        grid_spec=grid_spec,
        out_shape=jax.ShapeDtypeStruct((B, 1, OUT_F), jnp.float32),
    )(node_obs, adj, wp, bp, vc, W1, b1[None, :])
    return out.reshape(B, OUT_F)


# speedup vs baseline: 1.2899x; 1.0370x over previous
import functools

import jax
import jax.numpy as jnp
from jax.experimental import pallas as pl
from jax.experimental.pallas import tpu as pltpu

B, N, IN_F, OUT_F = 16, 1024, 64, 64
SENS = 0.05


def _gnn_batch_kernel(x_ref, a_ref, wp_ref, bp_ref, v_ref, w1_ref, b1_ref,
                      out_ref):
    x = x_ref[0]
    a = a_ref[0]

    p = jax.lax.dot_general(x, wp_ref[...], (((1,), (0,)), ((), ())),
                            preferred_element_type=jnp.float32,
                            precision=jax.lax.Precision.DEFAULT) + bp_ref[...]
    t2a = p[:, 0:2 * OUT_F]
    t3 = p[:, 2 * OUT_F:3 * OUT_F]
    t4 = p[:, 3 * OUT_F:4 * OUT_F]

    u_row = jax.lax.dot_general(v_ref[:, :IN_F], x, (((1,), (1,)), ((), ())),
                                preferred_element_type=jnp.float32,
                                precision=jax.lax.Precision.DEFAULT) \
        + v_ref[:, IN_F:IN_F + 1]

    neg_inf = jnp.float32(-jnp.inf)
    contrib = None
    half = N // 2
    for h in range(2):
        lo, hi = h * half, (h + 1) * half
        t3_h = t3[lo:hi, :]
        a_h = a[:, lo:hi]
        u_h = u_row[:, lo:hi]

        d_h = jax.lax.dot_general(t4, t3_h, (((1,), (1,)), ((), ())),
                                  preferred_element_type=jnp.float32,
                                  precision=jax.lax.Precision.DEFAULT)

        mask_h = (a_h < SENS) & (a_h > 0)
        sm_h = jnp.where(mask_h, d_h + a_h * u_h, neg_inf)
        smax_h = jnp.max(sm_h, axis=0, keepdims=True)
        m_h = jnp.maximum(smax_h, jnp.float32(0.0))
        e_h = jnp.exp2(sm_h - m_h)

        ga_h = jax.lax.dot_general(e_h, t2a, (((0,), (0,)), ((), ())),
                                   preferred_element_type=jnp.float32,
                                   precision=jax.lax.Precision.DEFAULT)
        denom_h = ga_h[:, OUT_F:OUT_F + 1]
        invd_h = jnp.float32(1.0) / (denom_h + jnp.float32(1e-16))
        c_h = jax.lax.dot_general(invd_h, ga_h[:, 0:OUT_F],
                                  (((0,), (0,)), ((), ())),
                                  preferred_element_type=jnp.float32,
                                  precision=jax.lax.Precision.DEFAULT)
        contrib = c_h if contrib is None else contrib + c_h

    mean_x = jnp.mean(x, axis=0, keepdims=True)
    lin = jax.lax.dot_general(mean_x, w1_ref[...], (((1,), (1,)), ((), ())),
                              preferred_element_type=jnp.float32,
                              precision=jax.lax.Precision.HIGHEST) + b1_ref[...]
    out_ref[0, 0] = lin[0] + contrib[0] * (jnp.float32(1.0) / jnp.float32(N))


def kernel(node_obs, adj, W1, b1, W2, b2, W3, b3, W4, b4, W5, b5):
    log2e = jnp.float32(1.4426950408889634)
    ls = log2e / jnp.sqrt(jnp.float32(OUT_F))
    w5c = W5[:, 0]
    v = (W3.T @ w5c) * ls
    c = jnp.dot(b3, w5c) * ls
    zcols = jnp.zeros((IN_F, OUT_F), jnp.float32)
    wp = jnp.concatenate([W2.T, zcols, W3.T, W4.T * ls], axis=1)
    ones_bias = jnp.zeros((OUT_F,), jnp.float32).at[0].set(1.0)
    bp = jnp.concatenate([b2, ones_bias, b3, b4 * ls])[None, :]
    vc = jnp.concatenate([v, jnp.full((1,), c, jnp.float32),
                          jnp.zeros((IN_F - 1,), jnp.float32)])[None, :]

    grid_spec = pl.GridSpec(
        grid=(B,),
        in_specs=[
            pl.BlockSpec((1, N, IN_F), lambda b: (b, 0, 0)),
            pl.BlockSpec((1, N, N), lambda b: (b, 0, 0)),
            pl.BlockSpec((IN_F, 4 * OUT_F), lambda b: (0, 0)),
            pl.BlockSpec((1, 4 * OUT_F), lambda b: (0, 0)),
            pl.BlockSpec((1, 2 * IN_F), lambda b: (0, 0)),
            pl.BlockSpec((OUT_F, IN_F), lambda b: (0, 0)),
            pl.BlockSpec((1, OUT_F), lambda b: (0, 0)),
        ],
        out_specs=pl.BlockSpec((1, 1, OUT_F), lambda b: (b, 0, 0)),
    )

    out = pl.pallas_call(
        _gnn_batch_kernel,
        grid_spec=grid_spec,
        out_shape=jax.ShapeDtypeStruct((B, 1, OUT_F), jnp.float32),
    )(node_obs, adj, wp, bp, vc, W1, b1[None, :])
    return out.reshape(B, OUT_F)


# speedup vs baseline: 1.2981x; 1.0063x over previous
import functools

import jax
import jax.numpy as jnp
from jax.experimental import pallas as pl
from jax.experimental.pallas import tpu as pltpu

B, N, IN_F, OUT_F = 16, 1024, 64, 64
SENS = 0.05


def _gnn_batch_kernel(x_ref, a_ref, wp_ref, bp_ref, v_ref, w1_ref, b1_ref,
                      out_ref):
    x = x_ref[0]
    a = a_ref[0]

    p = jax.lax.dot_general(x, wp_ref[...], (((1,), (0,)), ((), ())),
                            preferred_element_type=jnp.float32,
                            precision=jax.lax.Precision.DEFAULT) + bp_ref[...]
    t2a = p[:, 0:2 * OUT_F]
    t3 = p[:, 2 * OUT_F:3 * OUT_F]
    t4 = p[:, 3 * OUT_F:4 * OUT_F]

    u_row = jax.lax.dot_general(v_ref[:, :IN_F], x, (((1,), (1,)), ((), ())),
                                preferred_element_type=jnp.float32,
                                precision=jax.lax.Precision.DEFAULT) \
        + v_ref[:, IN_F:IN_F + 1]

    neg_inf = jnp.float32(-jnp.inf)
    contrib = None
    half = N // 2
    for h in range(2):
        lo, hi = h * half, (h + 1) * half
        t3_h = t3[lo:hi, :]
        a_h = a[:, lo:hi]
        u_h = u_row[:, lo:hi]

        d_h = jax.lax.dot_general(t4, t3_h, (((1,), (1,)), ((), ())),
                                  preferred_element_type=jnp.float32,
                                  precision=jax.lax.Precision.DEFAULT)

        au = jax.lax.bitcast_convert_type(a_h, jnp.uint32)
        mask_h = (au - jnp.uint32(1)) < jnp.uint32(0x3D4CCCCD - 1)
        sm_h = jnp.where(mask_h, d_h + a_h * u_h, neg_inf)
        smax_h = jnp.max(sm_h, axis=0, keepdims=True)
        m_h = jnp.maximum(smax_h, jnp.float32(0.0))
        e_h = jnp.exp2(sm_h - m_h)

        ga_h = jax.lax.dot_general(e_h, t2a, (((0,), (0,)), ((), ())),
                                   preferred_element_type=jnp.float32,
                                   precision=jax.lax.Precision.DEFAULT)
        denom_h = ga_h[:, OUT_F:OUT_F + 1]
        invd_h = jnp.float32(1.0) / (denom_h + jnp.float32(1e-16))
        c_h = jax.lax.dot_general(invd_h, ga_h[:, 0:OUT_F],
                                  (((0,), (0,)), ((), ())),
                                  preferred_element_type=jnp.float32,
                                  precision=jax.lax.Precision.DEFAULT)
        contrib = c_h if contrib is None else contrib + c_h

    mean_x = jnp.mean(x, axis=0, keepdims=True)
    lin = jax.lax.dot_general(mean_x, w1_ref[...], (((1,), (1,)), ((), ())),
                              preferred_element_type=jnp.float32,
                              precision=jax.lax.Precision.HIGHEST) + b1_ref[...]
    out_ref[0, 0] = lin[0] + contrib[0] * (jnp.float32(1.0) / jnp.float32(N))


def kernel(node_obs, adj, W1, b1, W2, b2, W3, b3, W4, b4, W5, b5):
    log2e = jnp.float32(1.4426950408889634)
    ls = log2e / jnp.sqrt(jnp.float32(OUT_F))
    w5c = W5[:, 0]
    v = (W3.T @ w5c) * ls
    c = jnp.dot(b3, w5c) * ls
    zcols = jnp.zeros((IN_F, OUT_F), jnp.float32)
    wp = jnp.concatenate([W2.T, zcols, W3.T, W4.T * ls], axis=1)
    ones_bias = jnp.zeros((OUT_F,), jnp.float32).at[0].set(1.0)
    bp = jnp.concatenate([b2, ones_bias, b3, b4 * ls])[None, :]
    vc = jnp.concatenate([v, jnp.full((1,), c, jnp.float32),
                          jnp.zeros((IN_F - 1,), jnp.float32)])[None, :]

    grid_spec = pl.GridSpec(
        grid=(B,),
        in_specs=[
            pl.BlockSpec((1, N, IN_F), lambda b: (b, 0, 0)),
            pl.BlockSpec((1, N, N), lambda b: (b, 0, 0)),
            pl.BlockSpec((IN_F, 4 * OUT_F), lambda b: (0, 0)),
            pl.BlockSpec((1, 4 * OUT_F), lambda b: (0, 0)),
            pl.BlockSpec((1, 2 * IN_F), lambda b: (0, 0)),
            pl.BlockSpec((OUT_F, IN_F), lambda b: (0, 0)),
            pl.BlockSpec((1, OUT_F), lambda b: (0, 0)),
        ],
        out_specs=pl.BlockSpec((1, 1, OUT_F), lambda b: (b, 0, 0)),
    )

    out = pl.pallas_call(
        _gnn_batch_kernel,
        grid_spec=grid_spec,
        out_shape=jax.ShapeDtypeStruct((B, 1, OUT_F), jnp.float32),
    )(node_obs, adj, wp, bp, vc, W1, b1[None, :])
    return out.reshape(B, OUT_F)


# speedup vs baseline: 1.3014x; 1.0026x over previous
import jax
import jax.numpy as jnp
from jax.experimental import pallas as pl

B, N, IN_F, OUT_F = 16, 1024, 64, 64
SENS = 0.05


def _gnn_batch_kernel(x_ref, a_ref, wp_ref, bp_ref, v_ref, w1_ref, b1_ref,
                      out_ref):
    x = x_ref[0]
    a = a_ref[0]

    p = jax.lax.dot_general(x, wp_ref[...], (((1,), (0,)), ((), ())),
                            preferred_element_type=jnp.float32,
                            precision=jax.lax.Precision.DEFAULT) + bp_ref[...]
    t2a = p[:, 0:2 * OUT_F]
    t3 = p[:, 2 * OUT_F:3 * OUT_F]
    t4 = p[:, 3 * OUT_F:4 * OUT_F]

    u_row = jax.lax.dot_general(v_ref[:, :IN_F], x, (((1,), (1,)), ((), ())),
                                preferred_element_type=jnp.float32,
                                precision=jax.lax.Precision.DEFAULT) \
        + v_ref[:, IN_F:IN_F + 1]

    neg_inf = jnp.float32(-jnp.inf)
    contrib = None
    half = N // 2
    for h in range(2):
        lo, hi = h * half, (h + 1) * half
        t3_h = t3[lo:hi, :]
        a_h = a[:, lo:hi]
        u_h = u_row[:, lo:hi]

        d_h = jax.lax.dot_general(t4, t3_h, (((1,), (1,)), ((), ())),
                                  preferred_element_type=jnp.float32,
                                  precision=jax.lax.Precision.DEFAULT)

        au = jax.lax.bitcast_convert_type(a_h, jnp.uint32)
        mask_h = (au - jnp.uint32(1)) < jnp.uint32(0x3D4CCCCD - 1)
        sm_h = jnp.where(mask_h, d_h + a_h * u_h, neg_inf)
        smax_h = jnp.max(sm_h, axis=0, keepdims=True)
        m_h = jnp.maximum(smax_h, jnp.float32(0.0))
        e_h = jnp.exp2(sm_h - m_h)

        ga_h = jax.lax.dot_general(e_h, t2a, (((0,), (0,)), ((), ())),
                                   preferred_element_type=jnp.float32,
                                   precision=jax.lax.Precision.DEFAULT)
        denom_h = ga_h[:, OUT_F:OUT_F + 1]
        invd_h = jnp.float32(1.0) / (denom_h + jnp.float32(1e-16))
        c_h = jax.lax.dot_general(invd_h, ga_h[:, 0:OUT_F],
                                  (((0,), (0,)), ((), ())),
                                  preferred_element_type=jnp.float32,
                                  precision=jax.lax.Precision.DEFAULT)
        contrib = c_h if contrib is None else contrib + c_h

    mean_x = jnp.mean(x, axis=0, keepdims=True)
    lin = jax.lax.dot_general(mean_x, w1_ref[...], (((1,), (1,)), ((), ())),
                              preferred_element_type=jnp.float32,
                              precision=jax.lax.Precision.HIGHEST) + b1_ref[...]
    out_ref[0, 0] = lin[0] + contrib[0] * (jnp.float32(1.0) / jnp.float32(N))


def kernel(node_obs, adj, W1, b1, W2, b2, W3, b3, W4, b4, W5, b5):
    log2e = jnp.float32(1.4426950408889634)
    ls = log2e / jnp.sqrt(jnp.float32(OUT_F))
    w5c = W5[:, 0]
    v = (W3.T @ w5c) * ls
    c = jnp.dot(b3, w5c) * ls
    zcols = jnp.zeros((IN_F, OUT_F), jnp.float32)
    wp = jnp.concatenate([W2.T, zcols, W3.T, W4.T * ls], axis=1)
    ones_bias = jnp.zeros((OUT_F,), jnp.float32).at[0].set(1.0)
    bp = jnp.concatenate([b2, ones_bias, b3, b4 * ls])[None, :]
    vc = jnp.concatenate([v, jnp.full((1,), c, jnp.float32),
                          jnp.zeros((IN_F - 1,), jnp.float32)])[None, :]

    grid_spec = pl.GridSpec(
        grid=(B,),
        in_specs=[
            pl.BlockSpec((1, N, IN_F), lambda b: (b, 0, 0)),
            pl.BlockSpec((1, N, N), lambda b: (b, 0, 0)),
            pl.BlockSpec((IN_F, 4 * OUT_F), lambda b: (0, 0)),
            pl.BlockSpec((1, 4 * OUT_F), lambda b: (0, 0)),
            pl.BlockSpec((1, 2 * IN_F), lambda b: (0, 0)),
            pl.BlockSpec((OUT_F, IN_F), lambda b: (0, 0)),
            pl.BlockSpec((1, OUT_F), lambda b: (0, 0)),
        ],
        out_specs=pl.BlockSpec((1, 1, OUT_F), lambda b: (b, 0, 0)),
    )

    out = pl.pallas_call(
        _gnn_batch_kernel,
        grid_spec=grid_spec,
        out_shape=jax.ShapeDtypeStruct((B, 1, OUT_F), jnp.float32),
    )(node_obs, adj, wp, bp, vc, W1, b1[None, :])
    return out.reshape(B, OUT_F)
